# baseline (speedup 1.0000x reference)
import functools

import jax
import jax.numpy as jnp
from jax import lax
from jax.experimental import pallas as pl
from jax.experimental.pallas import tpu as pltpu
from jax.experimental.pallas import tpu_sc as plsc

_NC = 2
_NS = 16
_NBUF = 2
_KCH = 128
_ZCH = 80
_TRASH = 16


def _make_dst3(edge_index, n, e):
    nw = _NC * _NS
    ept = e // nw
    nch = -(-ept // _KCH)
    pad = nch * _KCH - ept
    dstv = edge_index[1].astype(jnp.int32)
    dst2 = dstv.reshape(nw, ept)
    tfill = n + (jnp.arange(nw * pad, dtype=jnp.int32) % _TRASH).reshape(nw, pad)
    dst3 = jnp.concatenate([dst2, tfill], axis=1).reshape(nw, nch, _KCH)
    last = jnp.concatenate(
        [n + (jnp.arange(pad, dtype=jnp.int32) % _TRASH),
         dstv[e - (_KCH - pad):]])
    return dst3.at[nw - 1, nch - 1].set(last)


def _sc_segment_partials(h, dst3, n_nodes):
    e, d = h.shape
    nw = _NC * _NS
    ept = e // nw
    nch = dst3.shape[1]
    nacc = n_nodes // _ZCH
    acc_iters = -(-nacc // _NS)
    acc_rows = n_nodes + _TRASH
    mesh = plsc.VectorSubcoreMesh(core_axis_name="core", subcore_axis_name="subcore")

    @functools.partial(
        pl.kernel,
        mesh=mesh,
        out_type=jax.ShapeDtypeStruct((_NC * n_nodes, d), jnp.float32),
        scratch_types=[
            pltpu.VMEM((nch, _KCH), jnp.int32),
            pltpu.VMEM((_NBUF, _KCH, d), jnp.float32),
            pltpu.VMEM_SHARED((acc_rows, d), jnp.float32),
            pltpu.SemaphoreType.DMA,
            pltpu.SemaphoreType.DMA,
            pltpu.SemaphoreType.DMA,
            pltpu.SemaphoreType.DMA,
            pltpu.SemaphoreType.DMA,
            pltpu.SemaphoreType.DMA,
        ],
    )
    def seg(h_hbm, dst_hbm, out_hbm, idx_v, rows_v, acc_sh,
            g0, g1, s0, s1, zsem, isem):
        c = lax.axis_index("core")
        s = lax.axis_index("subcore")
        wid = c * _NS + s
        gsem = (g0, g1)
        ssem = (s0, s1)
        eb = wid * ept

        def gather(j, r, sem):
            off = jnp.minimum(eb + j * _KCH, e - _KCH)
            return pltpu.make_async_copy(
                h_hbm.at[pl.ds(off, _KCH)], rows_v.at[r], sem)

        def scatter(j, r, sem):
            return pltpu.make_async_copy(
                rows_v.at[r], acc_sh.at[idx_v.at[j]], sem)

        zslab = rows_v.at[_NBUF - 1].at[pl.ds(0, _ZCH)]

        def zcopy(k):
            return pltpu.make_async_copy(
                zslab, acc_sh.at[pl.ds(k * _ZCH, _ZCH)], zsem)

        def tcopy():
            return pltpu.make_async_copy(
                rows_v.at[_NBUF - 1].at[pl.ds(0, _TRASH)],
                acc_sh.at[pl.ds(n_nodes, _TRASH)], zsem)

        pltpu.make_async_copy(dst_hbm.at[wid], idx_v, isem).start()

        def zrow(i, carry):
            def zlane(k, carry2):
                rows_v[_NBUF - 1, i, pl.ds(k * 16, 16)] = (
                    jnp.zeros((16,), jnp.float32))
                return carry2
            return lax.fori_loop(0, d // 16, zlane, carry)
        lax.fori_loop(0, _ZCH, zrow, 0)
        for r in range(acc_iters):
            k = r * _NS + s

            @pl.when(k < nacc)
            def _():
                zcopy(k).start()

        @pl.when(s == _NS - 1)
        def _():
            tcopy().start()
        for r in range(_NBUF - 1):
            gather(r, r, gsem[r]).start()
        for r in range(acc_iters):
            k = r * _NS + s

            @pl.when(k < nacc)
            def _():
                zcopy(k).wait()

        @pl.when(s == _NS - 1)
        def _():
            tcopy().wait()
        gather(_NBUF - 1, _NBUF - 1, gsem[_NBUF - 1]).start()
        pltpu.make_async_copy(dst_hbm.at[wid], idx_v, isem).wait()
        plsc.subcore_barrier()

        def body(jj, carry):
            j0 = _NBUF * jj
            for r in range(_NBUF):
                gather(j0 + r, r, gsem[r]).wait()
                scatter(j0 + r, r, ssem[r]).start(add=True)
            for r in range(_NBUF):
                jn = j0 + _NBUF + r
                scatter(j0 + r, r, ssem[r]).wait()

                @pl.when(jn < nch)
                def _():
                    gather(jn, r, gsem[r]).start()
            return carry
        lax.fori_loop(0, nch // _NBUF, body, 0)
        for j in range(nch - nch % _NBUF, nch):
            r = j % _NBUF
            gather(j, r, gsem[r]).wait()
            pltpu.sync_copy(rows_v.at[r], acc_sh.at[idx_v.at[j]], add=True)
        plsc.subcore_barrier()

        for r in range(acc_iters):
            k = r * _NS + s

            @pl.when(k < nacc)
            def _():
                pltpu.make_async_copy(
                    acc_sh.at[pl.ds(k * _ZCH, _ZCH)],
                    out_hbm.at[pl.ds(c * n_nodes + k * _ZCH, _ZCH)],
                    zsem).start()
        for r in range(acc_iters):
            k = r * _NS + s

            @pl.when(k < nacc)
            def _():
                pltpu.make_async_copy(
                    acc_sh.at[pl.ds(k * _ZCH, _ZCH)],
                    out_hbm.at[pl.ds(c * n_nodes + k * _ZCH, _ZCH)],
                    zsem).wait()

    return seg(h, dst3)


def _tc_body(x_ref, p0_ref, p1_ref, wxt_ref, wht_ref, b_ref, o_ref):
    acc = jnp.dot(x_ref[...], wxt_ref[...], preferred_element_type=jnp.float32)
    acc = acc + jnp.dot(p0_ref[...] + p1_ref[...], wht_ref[...],
                        preferred_element_type=jnp.float32)
    o_ref[...] = jnp.maximum(acc + b_ref[...], 0.0)


def _tc_linear_relu(x, parts, wxt, wht, b2):
    n, d = x.shape
    blk = 1000
    nblk = n // blk
    return pl.pallas_call(
        _tc_body,
        grid=(nblk,),
        in_specs=[
            pl.BlockSpec((blk, d), lambda i: (i, 0)),
            pl.BlockSpec((blk, d), lambda i: (i, 0)),
            pl.BlockSpec((blk, d), lambda i: (i + nblk, 0)),
            pl.BlockSpec((d, d), lambda i: (0, 0)),
            pl.BlockSpec((d, d), lambda i: (0, 0)),
            pl.BlockSpec((1, d), lambda i: (0, 0)),
        ],
        out_specs=pl.BlockSpec((blk, d), lambda i: (i, 0)),
        out_shape=jax.ShapeDtypeStruct((n, d), jnp.float32),
    )(x, parts, parts, wxt, wht, b2)


def kernel(x, h, edge_index, W, b):
    n, d = x.shape
    e = h.shape[0]
    dst3 = _make_dst3(edge_index, n, e)
    parts = _sc_segment_partials(h, dst3, n)
    wxt = W[:, :d].T
    wht = W[:, d:].T
    return _tc_linear_relu(x, parts, wxt, wht, b.reshape(1, d))


# speedup vs baseline: 1.1406x; 1.1406x over previous
import functools

import jax
import jax.numpy as jnp
from jax import lax
from jax.experimental import pallas as pl
from jax.experimental.pallas import tpu as pltpu
from jax.experimental.pallas import tpu_sc as plsc

_NC = 2
_NS = 16
_NBUF = 3
_CHUNK = 80


def _sc_segment_partials(h, dst_chunks, n_nodes):
    e, d = h.shape
    nw = _NC * _NS
    ept = e // nw
    nch = ept // _CHUNK
    nacc = n_nodes // _CHUNK
    acc_iters = -(-nacc // _NS)
    mesh = plsc.VectorSubcoreMesh(core_axis_name="core", subcore_axis_name="subcore")

    @functools.partial(
        pl.kernel,
        mesh=mesh,
        out_type=jax.ShapeDtypeStruct((_NC * n_nodes, d), jnp.float32),
        scratch_types=[
            pltpu.VMEM((nch, _CHUNK), jnp.int32),
            pltpu.VMEM((_NBUF, _CHUNK, d), jnp.float32),
            pltpu.VMEM_SHARED((n_nodes, d), jnp.float32),
            pltpu.SemaphoreType.DMA,
            pltpu.SemaphoreType.DMA,
            pltpu.SemaphoreType.DMA,
            pltpu.SemaphoreType.DMA,
            pltpu.SemaphoreType.DMA,
            pltpu.SemaphoreType.DMA,
            pltpu.SemaphoreType.DMA,
            pltpu.SemaphoreType.DMA,
        ],
    )
    def seg(h_hbm, dst_hbm, out_hbm, idx_v, rows_v, acc_sh,
            g0, g1, g2, s0, s1, s2, zsem, isem):
        c = lax.axis_index("core")
        s = lax.axis_index("subcore")
        wid = c * _NS + s
        gsem = (g0, g1, g2)
        ssem = (s0, s1, s2)
        zero_v = rows_v.at[_NBUF - 1]
        eb = wid * ept

        def gather(j, r, sem):
            return pltpu.make_async_copy(
                h_hbm.at[pl.ds(eb + j * _CHUNK, _CHUNK)], rows_v.at[r], sem)

        def scatter(j, r, sem):
            return pltpu.make_async_copy(
                rows_v.at[r], acc_sh.at[idx_v.at[j]], sem)

        def zcopy(k):
            return pltpu.make_async_copy(
                zero_v, acc_sh.at[pl.ds(k * _CHUNK, _CHUNK)], zsem)

        pltpu.make_async_copy(dst_hbm.at[wid], idx_v, isem).start()

        def zrow(i, carry):
            def zlane(k, carry2):
                rows_v[_NBUF - 1, i, pl.ds(k * 16, 16)] = (
                    jnp.zeros((16,), jnp.float32))
                return carry2
            return lax.fori_loop(0, d // 16, zlane, carry)
        lax.fori_loop(0, _CHUNK, zrow, 0)
        for r in range(acc_iters):
            k = r * _NS + s

            @pl.when(k < nacc)
            def _():
                zcopy(k).start()
        for r in range(_NBUF - 1):
            gather(r, r, gsem[r]).start()
        for r in range(acc_iters):
            k = r * _NS + s

            @pl.when(k < nacc)
            def _():
                zcopy(k).wait()
        gather(_NBUF - 1, _NBUF - 1, gsem[_NBUF - 1]).start()
        pltpu.make_async_copy(dst_hbm.at[wid], idx_v, isem).wait()
        plsc.subcore_barrier()

        def body(jj, carry):
            j0 = _NBUF * jj
            for r in range(_NBUF):
                gather(j0 + r, r, gsem[r]).wait()
                scatter(j0 + r, r, ssem[r]).start(add=True)
            for r in range(_NBUF):
                jn = j0 + _NBUF + r
                scatter(j0 + r, r, ssem[r]).wait()

                @pl.when(jn < nch)
                def _():
                    gather(jn, r, gsem[r]).start()
            return carry
        lax.fori_loop(0, nch // _NBUF, body, 0)
        for j in range(nch - nch % _NBUF, nch):
            r = j % _NBUF
            gather(j, r, gsem[r]).wait()
            pltpu.sync_copy(rows_v.at[r], acc_sh.at[idx_v.at[j]], add=True)
        plsc.subcore_barrier()

        for r in range(acc_iters):
            k = r * _NS + s

            @pl.when(k < nacc)
            def _():
                pltpu.make_async_copy(
                    acc_sh.at[pl.ds(k * _CHUNK, _CHUNK)],
                    out_hbm.at[pl.ds(c * n_nodes + k * _CHUNK, _CHUNK)],
                    zsem).start()
        for r in range(acc_iters):
            k = r * _NS + s

            @pl.when(k < nacc)
            def _():
                pltpu.make_async_copy(
                    acc_sh.at[pl.ds(k * _CHUNK, _CHUNK)],
                    out_hbm.at[pl.ds(c * n_nodes + k * _CHUNK, _CHUNK)],
                    zsem).wait()

    return seg(h, dst_chunks)


def _tc_body(x_ref, p0_ref, p1_ref, wx_ref, wh_ref, b_ref, o_ref):
    dnums = (((1,), (1,)), ((), ()))
    acc = lax.dot_general(x_ref[...], wx_ref[...], dnums,
                          preferred_element_type=jnp.float32)
    acc = acc + lax.dot_general(p0_ref[...] + p1_ref[...], wh_ref[...], dnums,
                                preferred_element_type=jnp.float32)
    o_ref[...] = jnp.maximum(acc + b_ref[...], 0.0)


def _tc_linear_relu(x, parts, W, b2):
    n, d = x.shape
    blk = 1000
    nblk = n // blk
    return pl.pallas_call(
        _tc_body,
        grid=(nblk,),
        in_specs=[
            pl.BlockSpec((blk, d), lambda i: (i, 0)),
            pl.BlockSpec((blk, d), lambda i: (i, 0)),
            pl.BlockSpec((blk, d), lambda i: (i + nblk, 0)),
            pl.BlockSpec((d, d), lambda i: (0, 0)),
            pl.BlockSpec((d, d), lambda i: (0, 1)),
            pl.BlockSpec((1, d), lambda i: (0, 0)),
        ],
        out_specs=pl.BlockSpec((blk, d), lambda i: (i, 0)),
        out_shape=jax.ShapeDtypeStruct((n, d), jnp.float32),
    )(x, parts, parts, W, W, b2)


def kernel(x, h, edge_index, W, b):
    n, d = x.shape
    e = h.shape[0]
    nw = _NC * _NS
    dst = edge_index[1].astype(jnp.int32).reshape(nw, e // (nw * _CHUNK), _CHUNK)
    parts = _sc_segment_partials(h, dst, n)
    return _tc_linear_relu(x, parts, W, b.reshape(1, d))


# speedup vs baseline: 1.1432x; 1.0022x over previous
import functools

import jax
import jax.numpy as jnp
from jax import lax
from jax.experimental import pallas as pl
from jax.experimental.pallas import tpu as pltpu
from jax.experimental.pallas import tpu_sc as plsc

_NC = 2
_NS = 16
_NBUF = 3
_CHUNK = 80


def _sc_segment_partials(h, dst_chunks, n_nodes):
    e, d = h.shape
    nw = _NC * _NS
    ept = e // nw
    nch = ept // _CHUNK
    nacc = n_nodes // _CHUNK
    acc_iters = -(-nacc // _NS)
    mesh = plsc.VectorSubcoreMesh(core_axis_name="core", subcore_axis_name="subcore")

    @functools.partial(
        pl.kernel,
        mesh=mesh,
        out_type=jax.ShapeDtypeStruct((_NC * n_nodes, d), jnp.float32),
        scratch_types=[
            pltpu.VMEM((nch, _CHUNK), jnp.int32),
            pltpu.VMEM((_NBUF, _CHUNK, d), jnp.float32),
            pltpu.VMEM_SHARED((n_nodes, d), jnp.float32),
            pltpu.SemaphoreType.DMA,
            pltpu.SemaphoreType.DMA,
            pltpu.SemaphoreType.DMA,
            pltpu.SemaphoreType.DMA,
            pltpu.SemaphoreType.DMA,
            pltpu.SemaphoreType.DMA,
            pltpu.SemaphoreType.DMA,
            pltpu.SemaphoreType.DMA,
        ],
    )
    def seg(h_hbm, dst_hbm, out_hbm, idx_v, rows_v, acc_sh,
            g0, g1, g2, s0, s1, s2, zsem, isem):
        c = lax.axis_index("core")
        s = lax.axis_index("subcore")
        wid = c * _NS + s
        gsem = (g0, g1, g2)
        ssem = (s0, s1, s2)
        zero_v = rows_v.at[_NBUF - 1]
        eb = wid * ept

        def gather(j, r, sem):
            return pltpu.make_async_copy(
                h_hbm.at[pl.ds(eb + j * _CHUNK, _CHUNK)], rows_v.at[r], sem)

        def scatter(j, r, sem):
            return pltpu.make_async_copy(
                rows_v.at[r], acc_sh.at[idx_v.at[j]], sem)

        def zcopy(k):
            return pltpu.make_async_copy(
                zero_v, acc_sh.at[pl.ds(k * _CHUNK, _CHUNK)], zsem)

        pltpu.make_async_copy(dst_hbm.at[wid], idx_v, isem).start()

        def zrow(i, carry):
            def zlane(k, carry2):
                rows_v[_NBUF - 1, i, pl.ds(k * 16, 16)] = (
                    jnp.zeros((16,), jnp.float32))
                return carry2
            return lax.fori_loop(0, d // 16, zlane, carry)
        lax.fori_loop(0, _CHUNK, zrow, 0)
        for r in range(acc_iters):
            k = r * _NS + s

            @pl.when(k < nacc)
            def _():
                zcopy(k).start()
        for r in range(_NBUF - 1):
            gather(r, r, gsem[r]).start()
        for r in range(acc_iters):
            k = r * _NS + s

            @pl.when(k < nacc)
            def _():
                zcopy(k).wait()
        gather(_NBUF - 1, _NBUF - 1, gsem[_NBUF - 1]).start()
        pltpu.make_async_copy(dst_hbm.at[wid], idx_v, isem).wait()
        plsc.subcore_barrier()

        def body(jj, carry):
            j0 = _NBUF * jj
            for r in range(_NBUF):
                gather(j0 + r, r, gsem[r]).wait()
                scatter(j0 + r, r, ssem[r]).start(add=True)
            for r in range(_NBUF):
                jn = j0 + _NBUF + r
                scatter(j0 + r, r, ssem[r]).wait()

                @pl.when(jn < nch)
                def _():
                    gather(jn, r, gsem[r]).start()
            return carry
        lax.fori_loop(0, nch // _NBUF, body, 0)
        for j in range(nch - nch % _NBUF, nch):
            r = j % _NBUF
            gather(j, r, gsem[r]).wait()
            pltpu.sync_copy(rows_v.at[r], acc_sh.at[idx_v.at[j]], add=True)
        plsc.subcore_barrier()

        for r in range(acc_iters):
            k = r * _NS + s

            @pl.when(k < nacc)
            def _():
                pltpu.make_async_copy(
                    acc_sh.at[pl.ds(k * _CHUNK, _CHUNK)],
                    out_hbm.at[pl.ds(c * n_nodes + k * _CHUNK, _CHUNK)],
                    zsem).start()
        for r in range(acc_iters):
            k = r * _NS + s

            @pl.when(k < nacc)
            def _():
                pltpu.make_async_copy(
                    acc_sh.at[pl.ds(k * _CHUNK, _CHUNK)],
                    out_hbm.at[pl.ds(c * n_nodes + k * _CHUNK, _CHUNK)],
                    zsem).wait()

    return seg(h, dst_chunks)


_DNUMS = (((1, ), (1,)), ((), ()))


def _tc_xw_body(x_ref, wx_ref, b_ref, o_ref):
    o_ref[...] = lax.dot_general(
        x_ref[...], wx_ref[...], _DNUMS,
        preferred_element_type=jnp.float32) + b_ref[...]


def _tc_xw(x, W, b2):
    n, d = x.shape
    blk = 2000
    return pl.pallas_call(
        _tc_xw_body,
        grid=(n // blk,),
        in_specs=[
            pl.BlockSpec((blk, d), lambda i: (i, 0)),
            pl.BlockSpec((d, d), lambda i: (0, 0)),
            pl.BlockSpec((1, d), lambda i: (0, 0)),
        ],
        out_specs=pl.BlockSpec((blk, d), lambda i: (i, 0)),
        out_shape=jax.ShapeDtypeStruct((n, d), jnp.float32),
    )(x, W, b2)


def _tc_out_body(xw_ref, p0_ref, p1_ref, wh_ref, o_ref):
    acc = lax.dot_general(p0_ref[...] + p1_ref[...], wh_ref[...], _DNUMS,
                          preferred_element_type=jnp.float32)
    o_ref[...] = jnp.maximum(acc + xw_ref[...], 0.0)


def _tc_out(xw, parts, W):
    n, d = xw.shape
    blk = 1000
    nblk = n // blk
    return pl.pallas_call(
        _tc_out_body,
        grid=(nblk,),
        in_specs=[
            pl.BlockSpec((blk, d), lambda i: (i, 0)),
            pl.BlockSpec((blk, d), lambda i: (i, 0)),
            pl.BlockSpec((blk, d), lambda i: (i + nblk, 0)),
            pl.BlockSpec((d, d), lambda i: (0, 1)),
        ],
        out_specs=pl.BlockSpec((blk, d), lambda i: (i, 0)),
        out_shape=jax.ShapeDtypeStruct((n, d), jnp.float32),
    )(xw, parts, parts, W)


def kernel(x, h, edge_index, W, b):
    n, d = x.shape
    e = h.shape[0]
    nw = _NC * _NS
    dst = edge_index[1].astype(jnp.int32).reshape(nw, e // (nw * _CHUNK), _CHUNK)
    parts = _sc_segment_partials(h, dst, n)
    xw = _tc_xw(x, W, b.reshape(1, d))
    return _tc_out(xw, parts, W)
